# baseline (speedup 1.0000x reference)
import functools

import jax
import jax.numpy as jnp
from jax import lax
from jax.experimental import pallas as pl
from jax.experimental.pallas import tpu as pltpu
from jax.experimental.pallas import tpu_sc as plsc

NU = 50000
NI = 50000
NT = NU + NI
D = 64
B = 4096
L = 16
NC = 2
NS = 16
CH = 128

HALF = NU // NC
ACC_ROWS = 25216
DUMP = 25088

_SC_PARAMS = pltpu.CompilerParams(
    needs_layout_passes=False, use_tc_tiling_on_sc=False)


def _zero_acc_slice(zeros_h, acc, s):
    pltpu.sync_copy(zeros_h, acc.at[pl.ds(s * 1576, 1576)])


def _scale_rows(gath, val_v):
    def body(e2, _):
        for u in range(2):
            e = e2 * 2 + u
            v16 = plsc.load_gather(val_v, [jnp.full((L,), e, jnp.int32)])
            for j in range(D // L):
                gath[e, pl.ds(j * L, L)] = gath[e, pl.ds(j * L, L)] * v16
        return 0

    lax.fori_loop(0, CH // 2, body, 0)


def _pipelined_edge_loop(s, ept, nch, row_h, col_h, val_h, x_h, acc,
                         bufs, prep):
    def issue_idx(i, b):
        off = s * ept + i * CH
        pltpu.async_copy(row_h.at[pl.ds(off, CH)], b[0], b[5])
        pltpu.async_copy(col_h.at[pl.ds(off, CH)], b[1], b[5])
        pltpu.async_copy(val_h.at[pl.ds(off, CH)], b[2], b[5])

    def wait_idx(i, b):
        off = s * ept + i * CH
        pltpu.make_async_copy(row_h.at[pl.ds(off, CH)], b[0], b[5]).wait()
        pltpu.make_async_copy(col_h.at[pl.ds(off, CH)], b[1], b[5]).wait()
        pltpu.make_async_copy(val_h.at[pl.ds(off, CH)], b[2], b[5]).wait()

    def issue_gather(b):
        pltpu.async_copy(x_h.at[b[1]], b[4], b[6])

    def wait_gather(b):
        pltpu.make_async_copy(x_h.at[b[1]], b[4], b[6]).wait()

    def issue_scatter(b):
        pltpu.async_copy(b[4], acc.at[b[3]], b[7], add=True)

    def wait_scatter(b):
        pltpu.make_async_copy(b[4], acc.at[b[3]], b[7]).wait()

    issue_idx(0, bufs[0])
    wait_idx(0, bufs[0])
    prep(bufs[0])
    issue_gather(bufs[0])
    issue_idx(1, bufs[1])

    def body(ii, _):
        for p in range(2):
            i = 2 * ii + p
            cur = bufs[p]
            nxt = bufs[1 - p]

            @pl.when(i + 1 < nch)
            def _():
                wait_idx(i + 1, nxt)

            @pl.when(i >= 1)
            def _():
                wait_scatter(nxt)

            @pl.when(i + 1 < nch)
            def _():
                prep(nxt)
                issue_gather(nxt)

            wait_gather(cur)
            _scale_rows(cur[4], cur[2])

            @pl.when(i + 2 < nch)
            def _():
                issue_idx(i + 2, cur)

            issue_scatter(cur)
        return 0

    lax.fori_loop(0, nch // 2, body, 0)
    wait_scatter(bufs[1])


def _edge_scratch():
    return [
        pltpu.VMEM((CH,), jnp.int32),
        pltpu.VMEM((CH,), jnp.int32),
        pltpu.VMEM((CH,), jnp.float32),
        pltpu.VMEM((CH,), jnp.int32),
        pltpu.VMEM((CH, D), jnp.float32),
        pltpu.SemaphoreType.DMA,
        pltpu.SemaphoreType.DMA,
        pltpu.SemaphoreType.DMA,
    ]


def _make_spmm_uu(n_edges_pad):
    ept = n_edges_pad // NS
    nch = ept // CH
    mesh = plsc.VectorSubcoreMesh(
        core_axis_name="c", subcore_axis_name="s", num_cores=NC, num_subcores=NS)

    @functools.partial(
        pl.kernel,
        out_type=jax.ShapeDtypeStruct((NU, D), jnp.float32),
        mesh=mesh,
        compiler_params=_SC_PARAMS,
        scratch_types=_edge_scratch() + _edge_scratch() + [
            pltpu.VMEM_SHARED((ACC_ROWS, D), jnp.float32),
        ],
    )
    def spmm(row_h, col_h, val_h, x_h, zeros_h, y_h, *scratch):
        bufs = (tuple(scratch[0:8]), tuple(scratch[8:16]))
        acc = scratch[16]
        c = lax.axis_index("c")
        s = lax.axis_index("s")
        cbase = c * HALF

        _zero_acc_slice(zeros_h, acc, s)
        plsc.subcore_barrier()

        def prep(b):
            for g in range(CH // L):
                sl = pl.ds(g * L, L)
                loc = b[0][sl] - cbase
                own = (loc >= 0) & (loc < HALF)
                b[3][sl] = jnp.where(own, loc, DUMP)

        _pipelined_edge_loop(s, ept, nch, row_h, col_h, val_h, x_h, acc,
                             bufs, prep)
        plsc.subcore_barrier()

        start = s * 1568

        @pl.when(s < 15)
        def _():
            pltpu.sync_copy(acc.at[pl.ds(start, 1568)],
                            y_h.at[pl.ds(cbase + start, 1568)])

        @pl.when(s == 15)
        def _():
            pltpu.sync_copy(acc.at[pl.ds(start, 1480)],
                            y_h.at[pl.ds(cbase + start, 1480)])

    return spmm


def _make_stage_b(n_edges_pad):
    ept = n_edges_pad // NS
    nch = ept // CH
    bpt = B // NS
    mesh = plsc.VectorSubcoreMesh(
        core_axis_name="c", subcore_axis_name="s", num_cores=NC, num_subcores=NS)

    @functools.partial(
        pl.kernel,
        out_type=jax.ShapeDtypeStruct((NC * B,), jnp.float32),
        mesh=mesh,
        compiler_params=_SC_PARAMS,
        scratch_types=_edge_scratch() + _edge_scratch() + [
            pltpu.VMEM((bpt,), jnp.int32),
            pltpu.VMEM((bpt,), jnp.int32),
            pltpu.VMEM((CH,), jnp.int32),
            pltpu.VMEM((CH, D), jnp.float32),
            pltpu.VMEM((bpt,), jnp.float32),
            pltpu.VMEM_SHARED((ACC_ROWS, D), jnp.float32),
        ],
    )
    def stage_b(row_h, col_h, val_h, light_h, zeros_h, users_h, items_h,
                gpart_h, *scratch):
        bufs = (tuple(scratch[0:8]), tuple(scratch[8:16]))
        users_v, items_v, iidx_v, vbuf, gout, acc = scratch[16:22]
        c = lax.axis_index("c")
        s = lax.axis_index("s")
        ibase = c * HALF
        ubuf = bufs[0][4]
        sem_g = bufs[0][6]

        _zero_acc_slice(zeros_h, acc, s)
        plsc.subcore_barrier()

        def prep(b):
            for g in range(CH // L):
                sl = pl.ds(g * L, L)
                c16 = b[1][sl]
                ok = c16 < NU
                b[1][sl] = jnp.where(ok, c16, 0)
                b[2][sl] = jnp.where(ok, b[2][sl], 0.0)
                loc = b[0][sl] - (NU + ibase)
                own = (loc >= 0) & (loc < HALF)
                b[3][sl] = jnp.where(own, loc, DUMP)

        _pipelined_edge_loop(s, ept, nch, row_h, col_h, val_h, light_h, acc,
                             bufs, prep)
        plsc.subcore_barrier()

        pltpu.sync_copy(users_h.at[pl.ds(s * bpt, bpt)], users_v)
        pltpu.sync_copy(items_h.at[pl.ds(s * bpt, bpt)], items_v)
        for half in range(bpt // CH):
            hsl = pl.ds(half * CH, CH)
            pltpu.async_copy(light_h.at[users_v.at[hsl]], ubuf, sem_g).wait()
            for g in range(CH // L):
                sl = pl.ds(half * CH + g * L, L)
                it16 = items_v[sl]
                loc = it16 - ibase
                own = (loc >= 0) & (loc < HALF)
                iidx_v[pl.ds(g * L, L)] = jnp.where(own, loc, DUMP)
            pltpu.sync_copy(acc.at[iidx_v], vbuf)
            for g in range(CH // L):
                rows16 = jnp.arange(g * L, (g + 1) * L, dtype=jnp.int32)

                def dot_body(d, a):
                    d16 = jnp.full((L,), d, jnp.int32)
                    u = plsc.load_gather(ubuf, [rows16, d16])
                    v = plsc.load_gather(vbuf, [rows16, d16])
                    return a + u * v

                acc16 = lax.fori_loop(0, D, dot_body,
                                      jnp.zeros((L,), jnp.float32))
                sl = pl.ds(half * CH + g * L, L)
                loc = items_v[sl] - ibase
                own = (loc >= 0) & (loc < HALF)
                gout[sl] = jnp.where(own, acc16, 0.0)
        pltpu.sync_copy(gout, gpart_h.at[pl.ds(c * B + s * bpt, bpt)])

    return stage_b


def _mean4(a, b, c, d):
    def body(a_r, b_r, c_r, d_r, o_r):
        o_r[...] = (a_r[...] + b_r[...] + c_r[...] + d_r[...]) * 0.25

    blk = 1000
    spec = pl.BlockSpec((blk, D), lambda i: (i, 0))
    return pl.pallas_call(
        body,
        out_shape=jax.ShapeDtypeStruct((NU, D), jnp.float32),
        grid=(NU // blk,),
        in_specs=[spec] * 4,
        out_specs=spec,
    )(a, b, c, d)


def _pad_edges(row, col, val, n_pad):
    n = row.shape[0]
    if n == n_pad:
        return row.astype(jnp.int32), col.astype(jnp.int32), val
    pz = n_pad - n
    row = jnp.concatenate([row.astype(jnp.int32), jnp.zeros((pz,), jnp.int32)])
    col = jnp.concatenate([col.astype(jnp.int32), jnp.zeros((pz,), jnp.int32)])
    val = jnp.concatenate([val, jnp.zeros((pz,), jnp.float32)])
    return row, col, val


@jax.jit
def kernel(users, items, uu_row, uu_col, uu_val, g_row, g_col, g_val, user_emb):
    n_uu = uu_row.shape[0]
    n_g = g_row.shape[0]
    unit = NS * CH * 2
    uu_pad = -(-n_uu // unit) * unit
    g_pad = -(-n_g // unit) * unit

    ur, uc, uv = _pad_edges(uu_row, uu_col, uu_val, uu_pad)
    gr, gc, gv = _pad_edges(g_row, g_col, g_val, g_pad)

    zeros_h = jnp.zeros((1576, D), jnp.float32)
    spmm = _make_spmm_uu(uu_pad)
    x0 = user_emb
    x1 = spmm(ur, uc, uv, x0, zeros_h)
    x2 = spmm(ur, uc, uv, x1, zeros_h)
    x3 = spmm(ur, uc, uv, x2, zeros_h)
    light = _mean4(x0, x1, x2, x3)

    stage_b = _make_stage_b(g_pad)
    gpart = stage_b(gr, gc, gv, light, zeros_h,
                    users.astype(jnp.int32), items.astype(jnp.int32))
    return gpart[:B] + gpart[B:]


# speedup vs baseline: 7.5669x; 7.5669x over previous
import functools

import jax
import jax.numpy as jnp
from jax import lax
from jax.experimental import pallas as pl
from jax.experimental.pallas import tpu as pltpu
from jax.experimental.pallas import tpu_sc as plsc

NU = 50000
NI = 50000
NT = NU + NI
D = 64
B = 4096
L = 16
NC = 2
NS = 16
CH = 128

HALF = NU // NC
ACC_ROWS = 25216
DUMP = 25088

_SC_PARAMS = pltpu.CompilerParams(
    needs_layout_passes=False, use_tc_tiling_on_sc=False)


def _zero_acc_slice(zeros_h, acc, s):
    pltpu.sync_copy(zeros_h, acc.at[pl.ds(s * 1576, 1576)])


def _scale_rows(gath, val_v):
    def body(e2, _):
        for u in range(2):
            e = e2 * 2 + u
            v16 = plsc.load_gather(val_v, [jnp.full((L,), e, jnp.int32)])
            for j in range(D // L):
                gath[e, pl.ds(j * L, L)] = gath[e, pl.ds(j * L, L)] * v16
        return 0

    lax.fori_loop(0, CH // 2, body, 0)


def _pipelined_edge_loop(s, ept, nch, row_h, col_h, val_h, x_h, acc,
                         bufs, prep):
    def issue_idx(i, b):
        off = s * ept + i * CH
        pltpu.async_copy(row_h.at[pl.ds(off, CH)], b[0], b[5])
        pltpu.async_copy(col_h.at[pl.ds(off, CH)], b[1], b[5])
        pltpu.async_copy(val_h.at[pl.ds(off, CH)], b[2], b[5])

    def wait_idx(i, b):
        off = s * ept + i * CH
        pltpu.make_async_copy(row_h.at[pl.ds(off, CH)], b[0], b[5]).wait()
        pltpu.make_async_copy(col_h.at[pl.ds(off, CH)], b[1], b[5]).wait()
        pltpu.make_async_copy(val_h.at[pl.ds(off, CH)], b[2], b[5]).wait()

    def issue_gather(b):
        pltpu.async_copy(x_h.at[b[1]], b[4], b[6])

    def wait_gather(b):
        pltpu.make_async_copy(x_h.at[b[1]], b[4], b[6]).wait()

    def issue_scatter(b):
        pltpu.async_copy(b[4], acc.at[b[3]], b[7], add=True)

    def wait_scatter(b):
        pltpu.make_async_copy(b[4], acc.at[b[3]], b[7]).wait()

    issue_idx(0, bufs[0])
    wait_idx(0, bufs[0])
    prep(bufs[0])
    issue_gather(bufs[0])
    issue_idx(1, bufs[1])

    def body(ii, _):
        for p in range(2):
            i = 2 * ii + p
            cur = bufs[p]
            nxt = bufs[1 - p]

            @pl.when(i + 1 < nch)
            def _():
                wait_idx(i + 1, nxt)

            @pl.when(i >= 1)
            def _():
                wait_scatter(nxt)

            @pl.when(i + 1 < nch)
            def _():
                prep(nxt)
                issue_gather(nxt)

            wait_gather(cur)
            _scale_rows(cur[4], cur[2])

            @pl.when(i + 2 < nch)
            def _():
                issue_idx(i + 2, cur)

            issue_scatter(cur)
        return 0

    lax.fori_loop(0, nch // 2, body, 0)
    wait_scatter(bufs[1])


def _edge_scratch():
    return [
        pltpu.VMEM((CH,), jnp.int32),
        pltpu.VMEM((CH,), jnp.int32),
        pltpu.VMEM((CH,), jnp.float32),
        pltpu.VMEM((CH,), jnp.int32),
        pltpu.VMEM((CH, D), jnp.float32),
        pltpu.SemaphoreType.DMA,
        pltpu.SemaphoreType.DMA,
        pltpu.SemaphoreType.DMA,
    ]


def _make_spmm_uu(n_edges_pad):
    ept = n_edges_pad // NS
    nch = ept // CH
    mesh = plsc.VectorSubcoreMesh(
        core_axis_name="c", subcore_axis_name="s", num_cores=NC, num_subcores=NS)

    @functools.partial(
        pl.kernel,
        out_type=jax.ShapeDtypeStruct((NU, D), jnp.float32),
        mesh=mesh,
        compiler_params=_SC_PARAMS,
        scratch_types=_edge_scratch() + _edge_scratch() + [
            pltpu.VMEM_SHARED((ACC_ROWS, D), jnp.float32),
        ],
    )
    def spmm(row_h, col_h, val_h, x_h, zeros_h, y_h, *scratch):
        bufs = (tuple(scratch[0:8]), tuple(scratch[8:16]))
        acc = scratch[16]
        c = lax.axis_index("c")
        s = lax.axis_index("s")
        cbase = c * HALF

        _zero_acc_slice(zeros_h, acc, s)
        plsc.subcore_barrier()

        def prep(b):
            for g in range(CH // L):
                sl = pl.ds(g * L, L)
                loc = b[0][sl] - cbase
                own = (loc >= 0) & (loc < HALF)
                b[3][sl] = jnp.where(own, loc, DUMP)

        _pipelined_edge_loop(s, ept, nch, row_h, col_h, val_h, x_h, acc,
                             bufs, prep)
        plsc.subcore_barrier()

        start = s * 1568

        @pl.when(s < 15)
        def _():
            pltpu.sync_copy(acc.at[pl.ds(start, 1568)],
                            y_h.at[pl.ds(cbase + start, 1568)])

        @pl.when(s == 15)
        def _():
            pltpu.sync_copy(acc.at[pl.ds(start, 1480)],
                            y_h.at[pl.ds(cbase + start, 1480)])

    return spmm


def _make_stage_b(n_edges_pad):
    ept = n_edges_pad // NS
    nch = ept // CH
    bpt = B // NS
    mesh = plsc.VectorSubcoreMesh(
        core_axis_name="c", subcore_axis_name="s", num_cores=NC, num_subcores=NS)

    @functools.partial(
        pl.kernel,
        out_type=jax.ShapeDtypeStruct((NC * B,), jnp.float32),
        mesh=mesh,
        compiler_params=_SC_PARAMS,
        scratch_types=_edge_scratch() + _edge_scratch() + [
            pltpu.VMEM((bpt,), jnp.int32),
            pltpu.VMEM((bpt,), jnp.int32),
            pltpu.VMEM((CH,), jnp.int32),
            pltpu.VMEM((CH, D), jnp.float32),
            pltpu.VMEM((bpt,), jnp.float32),
            pltpu.VMEM_SHARED((ACC_ROWS, D), jnp.float32),
        ],
    )
    def stage_b(row_h, col_h, val_h, light_h, zeros_h, users_h, items_h,
                gpart_h, *scratch):
        bufs = (tuple(scratch[0:8]), tuple(scratch[8:16]))
        users_v, items_v, iidx_v, vbuf, gout, acc = scratch[16:22]
        c = lax.axis_index("c")
        s = lax.axis_index("s")
        ibase = c * HALF
        ubuf = bufs[0][4]
        sem_g = bufs[0][6]

        _zero_acc_slice(zeros_h, acc, s)
        plsc.subcore_barrier()

        def prep(b):
            for g in range(CH // L):
                sl = pl.ds(g * L, L)
                c16 = b[1][sl]
                ok = c16 < NU
                fake = jnp.arange(g * L, (g + 1) * L, dtype=jnp.int32)
                b[1][sl] = jnp.where(ok, c16, fake)
                b[2][sl] = jnp.where(ok, b[2][sl], 0.0)
                loc = b[0][sl] - (NU + ibase)
                own = (loc >= 0) & (loc < HALF)
                b[3][sl] = jnp.where(own, loc, DUMP)

        _pipelined_edge_loop(s, ept, nch, row_h, col_h, val_h, light_h, acc,
                             bufs, prep)
        plsc.subcore_barrier()

        pltpu.sync_copy(users_h.at[pl.ds(s * bpt, bpt)], users_v)
        pltpu.sync_copy(items_h.at[pl.ds(s * bpt, bpt)], items_v)
        for half in range(bpt // CH):
            hsl = pl.ds(half * CH, CH)
            pltpu.async_copy(light_h.at[users_v.at[hsl]], ubuf, sem_g).wait()
            for g in range(CH // L):
                sl = pl.ds(half * CH + g * L, L)
                it16 = items_v[sl]
                loc = it16 - ibase
                own = (loc >= 0) & (loc < HALF)
                iidx_v[pl.ds(g * L, L)] = jnp.where(own, loc, DUMP)
            pltpu.sync_copy(acc.at[iidx_v], vbuf)
            for g in range(CH // L):
                rows16 = jnp.arange(g * L, (g + 1) * L, dtype=jnp.int32)

                def dot_body(d, a):
                    d16 = jnp.full((L,), d, jnp.int32)
                    u = plsc.load_gather(ubuf, [rows16, d16])
                    v = plsc.load_gather(vbuf, [rows16, d16])
                    return a + u * v

                acc16 = lax.fori_loop(0, D, dot_body,
                                      jnp.zeros((L,), jnp.float32))
                sl = pl.ds(half * CH + g * L, L)
                loc = items_v[sl] - ibase
                own = (loc >= 0) & (loc < HALF)
                gout[sl] = jnp.where(own, acc16, 0.0)
        pltpu.sync_copy(gout, gpart_h.at[pl.ds(c * B + s * bpt, bpt)])

    return stage_b


def _mean4(a, b, c, d):
    def body(a_r, b_r, c_r, d_r, o_r):
        o_r[...] = (a_r[...] + b_r[...] + c_r[...] + d_r[...]) * 0.25

    blk = 1000
    spec = pl.BlockSpec((blk, D), lambda i: (i, 0))
    return pl.pallas_call(
        body,
        out_shape=jax.ShapeDtypeStruct((NU, D), jnp.float32),
        grid=(NU // blk,),
        in_specs=[spec] * 4,
        out_specs=spec,
    )(a, b, c, d)


def _pad_edges(row, col, val, n_pad):
    n = row.shape[0]
    if n == n_pad:
        return row.astype(jnp.int32), col.astype(jnp.int32), val
    pz = n_pad - n
    row = jnp.concatenate([row.astype(jnp.int32), jnp.zeros((pz,), jnp.int32)])
    col = jnp.concatenate([col.astype(jnp.int32), jnp.zeros((pz,), jnp.int32)])
    val = jnp.concatenate([val, jnp.zeros((pz,), jnp.float32)])
    return row, col, val


@jax.jit
def kernel(users, items, uu_row, uu_col, uu_val, g_row, g_col, g_val, user_emb):
    n_uu = uu_row.shape[0]
    n_g = g_row.shape[0]
    unit = NS * CH * 2
    uu_pad = -(-n_uu // unit) * unit
    g_pad = -(-n_g // unit) * unit

    ur, uc, uv = _pad_edges(uu_row, uu_col, uu_val, uu_pad)
    gr, gc, gv = _pad_edges(g_row, g_col, g_val, g_pad)

    zeros_h = jnp.zeros((1576, D), jnp.float32)
    spmm = _make_spmm_uu(uu_pad)
    x0 = user_emb
    x1 = spmm(ur, uc, uv, x0, zeros_h)
    x2 = spmm(ur, uc, uv, x1, zeros_h)
    x3 = spmm(ur, uc, uv, x2, zeros_h)
    light = _mean4(x0, x1, x2, x3)

    stage_b = _make_stage_b(g_pad)
    gpart = stage_b(gr, gc, gv, light, zeros_h,
                    users.astype(jnp.int32), items.astype(jnp.int32))
    return gpart[:B] + gpart[B:]


# speedup vs baseline: 15.1349x; 2.0001x over previous
import functools

import jax
import jax.numpy as jnp
from jax import lax
from jax.experimental import pallas as pl
from jax.experimental.pallas import tpu as pltpu
from jax.experimental.pallas import tpu_sc as plsc

NU = 50000
NI = 50000
NT = NU + NI
D = 64
B = 4096
L = 16
NC = 2
NS = 16
NW = NC * NS
CH = 128

HALF = NU // NC
ACC_ROWS = 25216
DUMP = 25088

_SC_PARAMS = pltpu.CompilerParams(
    needs_layout_passes=False, use_tc_tiling_on_sc=False)

_MESH = dict(core_axis_name="c", subcore_axis_name="s",
             num_cores=NC, num_subcores=NS)


def _zero_acc_slice(zeros_h, acc, s):
    pltpu.sync_copy(zeros_h, acc.at[pl.ds(s * 1576, 1576)])


def _scale_rows(gath, val_v):
    def body(e2, _):
        for u in range(2):
            e = e2 * 2 + u
            v16 = plsc.load_gather(val_v, [jnp.full((L,), e, jnp.int32)])
            for j in range(D // L):
                gath[e, pl.ds(j * L, L)] = gath[e, pl.ds(j * L, L)] * v16
        return 0

    lax.fori_loop(0, CH // 2, body, 0)


def _make_partition(n_edges_pad, bipartite):
    ept = n_edges_pad // NS
    nch = ept // CH
    reg = ept + 2 * CH

    @functools.partial(
        pl.kernel,
        out_type=(
            jax.ShapeDtypeStruct((NW * reg,), jnp.int32),
            jax.ShapeDtypeStruct((NW * reg,), jnp.float32),
            jax.ShapeDtypeStruct((NW * reg,), jnp.int32),
            jax.ShapeDtypeStruct((NW * L,), jnp.int32),
        ),
        mesh=plsc.VectorSubcoreMesh(**_MESH),
        compiler_params=_SC_PARAMS,
        scratch_types=[
            pltpu.VMEM((CH,), jnp.int32),
            pltpu.VMEM((CH,), jnp.int32),
            pltpu.VMEM((CH,), jnp.float32),
            pltpu.SemaphoreType.DMA,
            pltpu.VMEM((CH,), jnp.int32),
            pltpu.VMEM((CH,), jnp.int32),
            pltpu.VMEM((CH,), jnp.float32),
            pltpu.SemaphoreType.DMA,
            pltpu.VMEM((2 * CH,), jnp.int32),
            pltpu.VMEM((2 * CH,), jnp.float32),
            pltpu.VMEM((2 * CH,), jnp.int32),
            pltpu.VMEM((L,), jnp.int32),
            pltpu.SemaphoreType.DMA,
        ],
    )
    def part(row_h, col_h, val_h, pcol_h, pval_h, psidx_h, counts_h,
             *scratch):
        bufs = (scratch[0:4], scratch[4:8])
        ccol, cval, csidx, cnt16, sem_f = scratch[8:13]
        c = lax.axis_index("c")
        s = lax.axis_index("s")
        widx = c * NS + s
        rbase = widx * reg
        cbase = (NU if bipartite else 0) + c * HALF

        def issue_idx(i, b):
            off = s * ept + i * CH
            pltpu.async_copy(row_h.at[pl.ds(off, CH)], b[0], b[3])
            pltpu.async_copy(col_h.at[pl.ds(off, CH)], b[1], b[3])
            pltpu.async_copy(val_h.at[pl.ds(off, CH)], b[2], b[3])

        def wait_idx(i, b):
            off = s * ept + i * CH
            pltpu.make_async_copy(row_h.at[pl.ds(off, CH)], b[0], b[3]).wait()
            pltpu.make_async_copy(col_h.at[pl.ds(off, CH)], b[1], b[3]).wait()
            pltpu.make_async_copy(val_h.at[pl.ds(off, CH)], b[2], b[3]).wait()

        def flush_descs(flushed, off):
            flushed = pl.multiple_of(flushed, CH)
            off = pl.multiple_of(off, CH)
            return (
                pltpu.make_async_copy(ccol.at[pl.ds(off, CH)],
                                      pcol_h.at[pl.ds(rbase + flushed, CH)],
                                      sem_f),
                pltpu.make_async_copy(cval.at[pl.ds(off, CH)],
                                      pval_h.at[pl.ds(rbase + flushed, CH)],
                                      sem_f),
                pltpu.make_async_copy(csidx.at[pl.ds(off, CH)],
                                      psidx_h.at[pl.ds(rbase + flushed, CH)],
                                      sem_f),
            )

        issue_idx(0, bufs[0])

        def body(ii, carry):
            nc, flushed = carry
            for p in range(2):
                i = 2 * ii + p
                cur = bufs[p]
                nxt = bufs[1 - p]
                wait_idx(i, cur)

                @pl.when(i + 1 < nch)
                def _():
                    issue_idx(i + 1, nxt)

                for g in range(CH // L):
                    sl = pl.ds(g * L, L)
                    r16 = cur[0][sl]
                    c16 = cur[1][sl]
                    v16 = cur[2][sl]
                    loc = r16 - cbase
                    m = (loc >= 0) & (loc < HALF)
                    if bipartite:
                        m = m & (c16 < NU)
                    mi = m.astype(jnp.int32)
                    cum = plsc.cumsum(mi)
                    pos = (jnp.full((L,), nc, jnp.int32) + cum - mi) & (2 * CH - 1)
                    plsc.store_scatter(ccol, [pos], c16, mask=m)
                    plsc.store_scatter(cval, [pos], v16, mask=m)
                    plsc.store_scatter(csidx, [pos], loc, mask=m)
                    nc = nc + jnp.sum(mi)

                do_flush = nc - flushed >= CH

                @pl.when(do_flush)
                def _():
                    @pl.when(flushed > 0)
                    def _():
                        for d in flush_descs(flushed - CH,
                                             (flushed - CH) & CH):
                            d.wait()
                    for d in flush_descs(flushed, flushed & CH):
                        d.start()

                flushed = jnp.where(do_flush, flushed + CH, flushed)
            return nc, flushed

        nc, flushed = lax.fori_loop(0, nch // 2, body,
                                    (jnp.int32(0), jnp.int32(0)))

        rem = nc - flushed
        flushed = pl.multiple_of(flushed, CH)
        off = pl.multiple_of(flushed & CH, CH)
        rem16 = jnp.full((L,), rem, jnp.int32)
        for g in range(CH // L):
            lane = jnp.arange(g * L, (g + 1) * L, dtype=jnp.int32)
            mf = lane >= rem16
            pos = jnp.full((L,), off, jnp.int32) + lane
            plsc.store_scatter(ccol, [pos], lane, mask=mf)
            plsc.store_scatter(cval, [pos], jnp.zeros((L,), jnp.float32),
                               mask=mf)
            plsc.store_scatter(csidx, [pos], jnp.full((L,), DUMP, jnp.int32),
                               mask=mf)

        @pl.when(flushed > 0)
        def _():
            for d in flush_descs(flushed - CH, (flushed - CH) & CH):
                d.wait()

        pltpu.sync_copy(ccol.at[pl.ds(off, CH)],
                        pcol_h.at[pl.ds(rbase + flushed, CH)])
        pltpu.sync_copy(cval.at[pl.ds(off, CH)],
                        pval_h.at[pl.ds(rbase + flushed, CH)])
        pltpu.sync_copy(csidx.at[pl.ds(off, CH)],
                        psidx_h.at[pl.ds(rbase + flushed, CH)])

        cnt16[...] = jnp.full((L,), nc, jnp.int32)
        pltpu.sync_copy(cnt16, counts_h.at[pl.ds(widx * L, L)])

    return part, reg


def _edge_scratch():
    return [
        pltpu.VMEM((CH,), jnp.int32),
        pltpu.VMEM((CH,), jnp.float32),
        pltpu.VMEM((CH,), jnp.int32),
        pltpu.VMEM((CH, D), jnp.float32),
        pltpu.SemaphoreType.DMA,
        pltpu.SemaphoreType.DMA,
        pltpu.SemaphoreType.DMA,
    ]


def _tile_chunk_count(counts_h, cnt16, widx):
    pltpu.sync_copy(counts_h.at[pl.ds(widx * L, L)], cnt16)
    cnt = jnp.max(cnt16[...])
    return jnp.maximum((cnt + CH - 1) >> 7, 1)


def _compact_edge_loop(rbase, nct, nch_max, pcol_h, pval_h, psidx_h,
                       x_h, acc, bufs):
    def issue_idx(i, b):
        off = rbase + i * CH
        pltpu.async_copy(pcol_h.at[pl.ds(off, CH)], b[0], b[4])
        pltpu.async_copy(pval_h.at[pl.ds(off, CH)], b[1], b[4])
        pltpu.async_copy(psidx_h.at[pl.ds(off, CH)], b[2], b[4])

    def wait_idx(i, b):
        off = rbase + i * CH
        pltpu.make_async_copy(pcol_h.at[pl.ds(off, CH)], b[0], b[4]).wait()
        pltpu.make_async_copy(pval_h.at[pl.ds(off, CH)], b[1], b[4]).wait()
        pltpu.make_async_copy(psidx_h.at[pl.ds(off, CH)], b[2], b[4]).wait()

    def issue_gather(b):
        pltpu.async_copy(x_h.at[b[0]], b[3], b[5])

    def wait_gather(b):
        pltpu.make_async_copy(x_h.at[b[0]], b[3], b[5]).wait()

    def issue_scatter(b):
        pltpu.async_copy(b[3], acc.at[b[2]], b[6], add=True)

    def wait_scatter(b):
        pltpu.make_async_copy(b[3], acc.at[b[2]], b[6]).wait()

    issue_idx(0, bufs[0])
    wait_idx(0, bufs[0])
    issue_gather(bufs[0])

    @pl.when(1 < nct)
    def _():
        issue_idx(1, bufs[1])

    def body(ii, _):
        for p in range(2):
            i = 2 * ii + p
            cur = bufs[p]
            nxt = bufs[1 - p]

            @pl.when(i < nct)
            def _():
                @pl.when(i + 1 < nct)
                def _():
                    wait_idx(i + 1, nxt)

                @pl.when(i >= 1)
                def _():
                    wait_scatter(nxt)

                @pl.when(i + 1 < nct)
                def _():
                    issue_gather(nxt)

                wait_gather(cur)
                _scale_rows(cur[3], cur[1])

                @pl.when(i + 2 < nct)
                def _():
                    issue_idx(i + 2, cur)

                issue_scatter(cur)
        return 0

    lax.fori_loop(0, nch_max // 2, body, 0)

    @pl.when((nct & 1) == 1)
    def _():
        wait_scatter(bufs[0])

    @pl.when((nct & 1) == 0)
    def _():
        wait_scatter(bufs[1])


def _make_spmm_compact(reg):
    nch_max = reg // CH

    @functools.partial(
        pl.kernel,
        out_type=jax.ShapeDtypeStruct((NU, D), jnp.float32),
        mesh=plsc.VectorSubcoreMesh(**_MESH),
        compiler_params=_SC_PARAMS,
        scratch_types=_edge_scratch() + _edge_scratch() + [
            pltpu.VMEM((L,), jnp.int32),
            pltpu.VMEM_SHARED((ACC_ROWS, D), jnp.float32),
        ],
    )
    def spmm(pcol_h, pval_h, psidx_h, counts_h, x_h, zeros_h, y_h, *scratch):
        bufs = (tuple(scratch[0:7]), tuple(scratch[7:14]))
        cnt16, acc = scratch[14], scratch[15]
        c = lax.axis_index("c")
        s = lax.axis_index("s")
        widx = c * NS + s

        _zero_acc_slice(zeros_h, acc, s)
        plsc.subcore_barrier()

        nct = _tile_chunk_count(counts_h, cnt16, widx)
        _compact_edge_loop(widx * reg, nct, nch_max,
                           pcol_h, pval_h, psidx_h, x_h, acc, bufs)
        plsc.subcore_barrier()

        start = s * 1568
        cbase = c * HALF

        @pl.when(s < 15)
        def _():
            pltpu.sync_copy(acc.at[pl.ds(start, 1568)],
                            y_h.at[pl.ds(cbase + start, 1568)])

        @pl.when(s == 15)
        def _():
            pltpu.sync_copy(acc.at[pl.ds(start, 1480)],
                            y_h.at[pl.ds(cbase + start, 1480)])

    return spmm


def _make_stage_b(reg):
    nch_max = reg // CH
    bpt = B // NS

    @functools.partial(
        pl.kernel,
        out_type=jax.ShapeDtypeStruct((NC * B,), jnp.float32),
        mesh=plsc.VectorSubcoreMesh(**_MESH),
        compiler_params=_SC_PARAMS,
        scratch_types=_edge_scratch() + _edge_scratch() + [
            pltpu.VMEM((L,), jnp.int32),
            pltpu.VMEM((bpt,), jnp.int32),
            pltpu.VMEM((bpt,), jnp.int32),
            pltpu.VMEM((CH,), jnp.int32),
            pltpu.VMEM((CH, D), jnp.float32),
            pltpu.VMEM((bpt,), jnp.float32),
            pltpu.VMEM_SHARED((ACC_ROWS, D), jnp.float32),
        ],
    )
    def stage_b(pcol_h, pval_h, psidx_h, counts_h, light_h, zeros_h,
                users_h, items_h, gpart_h, *scratch):
        bufs = (tuple(scratch[0:7]), tuple(scratch[7:14]))
        cnt16, users_v, items_v, iidx_v, vbuf, gout, acc = scratch[14:21]
        c = lax.axis_index("c")
        s = lax.axis_index("s")
        widx = c * NS + s
        ibase = c * HALF
        ubuf = bufs[0][3]
        sem_g = bufs[0][5]

        _zero_acc_slice(zeros_h, acc, s)
        plsc.subcore_barrier()

        nct = _tile_chunk_count(counts_h, cnt16, widx)
        _compact_edge_loop(widx * reg, nct, nch_max,
                           pcol_h, pval_h, psidx_h, light_h, acc, bufs)
        plsc.subcore_barrier()

        pltpu.sync_copy(users_h.at[pl.ds(s * bpt, bpt)], users_v)
        pltpu.sync_copy(items_h.at[pl.ds(s * bpt, bpt)], items_v)
        for half in range(bpt // CH):
            hsl = pl.ds(half * CH, CH)
            pltpu.async_copy(light_h.at[users_v.at[hsl]], ubuf, sem_g).wait()
            for g in range(CH // L):
                sl = pl.ds(half * CH + g * L, L)
                loc = items_v[sl] - ibase
                own = (loc >= 0) & (loc < HALF)
                fake = jnp.arange(g * L, (g + 1) * L, dtype=jnp.int32)
                iidx_v[pl.ds(g * L, L)] = jnp.where(own, loc, DUMP + fake)
            pltpu.sync_copy(acc.at[iidx_v], vbuf)
            for g in range(CH // L):
                rows16 = jnp.arange(g * L, (g + 1) * L, dtype=jnp.int32)

                def dot_body(d, a):
                    d16 = jnp.full((L,), d, jnp.int32)
                    u = plsc.load_gather(ubuf, [rows16, d16])
                    v = plsc.load_gather(vbuf, [rows16, d16])
                    return a + u * v

                acc16 = lax.fori_loop(0, D, dot_body,
                                      jnp.zeros((L,), jnp.float32))
                sl = pl.ds(half * CH + g * L, L)
                loc = items_v[sl] - ibase
                own = (loc >= 0) & (loc < HALF)
                gout[sl] = jnp.where(own, acc16, 0.0)
        pltpu.sync_copy(gout, gpart_h.at[pl.ds(c * B + s * bpt, bpt)])

    return stage_b


def _mean4(a, b, c, d):
    def body(a_r, b_r, c_r, d_r, o_r):
        o_r[...] = (a_r[...] + b_r[...] + c_r[...] + d_r[...]) * 0.25

    blk = 1000
    spec = pl.BlockSpec((blk, D), lambda i: (i, 0))
    return pl.pallas_call(
        body,
        out_shape=jax.ShapeDtypeStruct((NU, D), jnp.float32),
        grid=(NU // blk,),
        in_specs=[spec] * 4,
        out_specs=spec,
    )(a, b, c, d)


def _pad_edges(row, col, val, n_pad):
    n = row.shape[0]
    if n == n_pad:
        return row.astype(jnp.int32), col.astype(jnp.int32), val
    pz = n_pad - n
    row = jnp.concatenate([row.astype(jnp.int32), jnp.zeros((pz,), jnp.int32)])
    col = jnp.concatenate([col.astype(jnp.int32),
                           jnp.arange(pz, dtype=jnp.int32) % NU])
    val = jnp.concatenate([val, jnp.zeros((pz,), jnp.float32)])
    return row, col, val


@jax.jit
def kernel(users, items, uu_row, uu_col, uu_val, g_row, g_col, g_val, user_emb):
    n_uu = uu_row.shape[0]
    n_g = g_row.shape[0]
    unit = NS * CH * 2
    uu_pad = -(-n_uu // unit) * unit
    g_pad = -(-n_g // unit) * unit

    ur, uc, uv = _pad_edges(uu_row, uu_col, uu_val, uu_pad)
    gr, gc, gv = _pad_edges(g_row, g_col, g_val, g_pad)

    zeros_h = jnp.zeros((1576, D), jnp.float32)

    part_uu, reg_uu = _make_partition(uu_pad, bipartite=False)
    pcol_u, pval_u, psidx_u, cnt_u = part_uu(ur, uc, uv)

    spmm = _make_spmm_compact(reg_uu)
    x0 = user_emb
    x1 = spmm(pcol_u, pval_u, psidx_u, cnt_u, x0, zeros_h)
    x2 = spmm(pcol_u, pval_u, psidx_u, cnt_u, x1, zeros_h)
    x3 = spmm(pcol_u, pval_u, psidx_u, cnt_u, x2, zeros_h)
    light = _mean4(x0, x1, x2, x3)

    part_g, reg_g = _make_partition(g_pad, bipartite=True)
    pcol_g, pval_g, psidx_g, cnt_g = part_g(gr, gc, gv)

    stage_b = _make_stage_b(reg_g)
    gpart = stage_b(pcol_g, pval_g, psidx_g, cnt_g, light, zeros_h,
                    users.astype(jnp.int32), items.astype(jnp.int32))
    return gpart[:B] + gpart[B:]


# speedup vs baseline: 20.2707x; 1.3393x over previous
import functools

import jax
import jax.numpy as jnp
from jax import lax
from jax.experimental import pallas as pl
from jax.experimental.pallas import tpu as pltpu
from jax.experimental.pallas import tpu_sc as plsc

NU = 50000
NI = 50000
NT = NU + NI
D = 64
B = 4096
L = 16
NC = 2
NS = 16
NW = NC * NS
CH = 128

HALF = NU // NC
ACC_ROWS = 25216
DUMP = 25088

_SC_PARAMS = pltpu.CompilerParams(
    needs_layout_passes=False, use_tc_tiling_on_sc=False)

_MESH = dict(core_axis_name="c", subcore_axis_name="s",
             num_cores=NC, num_subcores=NS)


def _zero_acc_slice(zeros_h, acc, s):
    pltpu.sync_copy(zeros_h, acc.at[pl.ds(s * 1576, 1576)])


def _scale_rows(gath, val_v):
    @functools.partial(plsc.parallel_loop, 0, CH, unroll=4)
    def _(e):
        v16 = plsc.load_gather(val_v, [jnp.full((L,), e, jnp.int32)])
        for j in range(D // L):
            gath[e, pl.ds(j * L, L)] = gath[e, pl.ds(j * L, L)] * v16


def _make_partition(n_edges_pad, bipartite):
    ept = n_edges_pad // NS
    nch = ept // CH
    reg = ept + 2 * CH

    @functools.partial(
        pl.kernel,
        out_type=(
            jax.ShapeDtypeStruct((NW * reg,), jnp.int32),
            jax.ShapeDtypeStruct((NW * reg,), jnp.float32),
            jax.ShapeDtypeStruct((NW * reg,), jnp.int32),
            jax.ShapeDtypeStruct((NW * L,), jnp.int32),
        ),
        mesh=plsc.VectorSubcoreMesh(**_MESH),
        compiler_params=_SC_PARAMS,
        scratch_types=[
            pltpu.VMEM((CH,), jnp.int32),
            pltpu.VMEM((CH,), jnp.int32),
            pltpu.VMEM((CH,), jnp.float32),
            pltpu.SemaphoreType.DMA,
            pltpu.VMEM((CH,), jnp.int32),
            pltpu.VMEM((CH,), jnp.int32),
            pltpu.VMEM((CH,), jnp.float32),
            pltpu.SemaphoreType.DMA,
            pltpu.VMEM((2 * CH,), jnp.int32),
            pltpu.VMEM((2 * CH,), jnp.float32),
            pltpu.VMEM((2 * CH,), jnp.int32),
            pltpu.VMEM((L,), jnp.int32),
            pltpu.SemaphoreType.DMA,
        ],
    )
    def part(row_h, col_h, val_h, pcol_h, pval_h, psidx_h, counts_h,
             *scratch):
        bufs = (scratch[0:4], scratch[4:8])
        ccol, cval, csidx, cnt16, sem_f = scratch[8:13]
        c = lax.axis_index("c")
        s = lax.axis_index("s")
        widx = c * NS + s
        rbase = widx * reg
        cbase = (NU if bipartite else 0) + c * HALF

        def issue_idx(i, b):
            off = s * ept + i * CH
            pltpu.async_copy(row_h.at[pl.ds(off, CH)], b[0], b[3])
            pltpu.async_copy(col_h.at[pl.ds(off, CH)], b[1], b[3])
            pltpu.async_copy(val_h.at[pl.ds(off, CH)], b[2], b[3])

        def wait_idx(i, b):
            off = s * ept + i * CH
            pltpu.make_async_copy(row_h.at[pl.ds(off, CH)], b[0], b[3]).wait()
            pltpu.make_async_copy(col_h.at[pl.ds(off, CH)], b[1], b[3]).wait()
            pltpu.make_async_copy(val_h.at[pl.ds(off, CH)], b[2], b[3]).wait()

        def flush_descs(flushed, off):
            flushed = pl.multiple_of(flushed, CH)
            off = pl.multiple_of(off, CH)
            return (
                pltpu.make_async_copy(ccol.at[pl.ds(off, CH)],
                                      pcol_h.at[pl.ds(rbase + flushed, CH)],
                                      sem_f),
                pltpu.make_async_copy(cval.at[pl.ds(off, CH)],
                                      pval_h.at[pl.ds(rbase + flushed, CH)],
                                      sem_f),
                pltpu.make_async_copy(csidx.at[pl.ds(off, CH)],
                                      psidx_h.at[pl.ds(rbase + flushed, CH)],
                                      sem_f),
            )

        issue_idx(0, bufs[0])

        def body(ii, carry):
            nc16, flushed = carry
            for p in range(2):
                i = 2 * ii + p
                cur = bufs[p]
                nxt = bufs[1 - p]
                wait_idx(i, cur)

                @pl.when(i + 1 < nch)
                def _():
                    issue_idx(i + 1, nxt)

                for g in range(CH // L):
                    sl = pl.ds(g * L, L)
                    r16 = cur[0][sl]
                    c16 = cur[1][sl]
                    v16 = cur[2][sl]
                    loc = r16 - cbase
                    m = (loc >= 0) & (loc < HALF)
                    if bipartite:
                        m = m & (c16 < NU)
                    mi = m.astype(jnp.int32)
                    cum = plsc.cumsum(mi)
                    pos = (nc16 + cum - mi) & (2 * CH - 1)
                    plsc.store_scatter(ccol, [pos], c16, mask=m)
                    plsc.store_scatter(cval, [pos], v16, mask=m)
                    plsc.store_scatter(csidx, [pos], loc, mask=m)
                    nc16 = nc16 + plsc.all_reduce_population_count(m)

                nc = jnp.max(nc16)
                do_flush = nc - flushed >= CH

                @pl.when(do_flush)
                def _():
                    @pl.when(flushed > 0)
                    def _():
                        for d in flush_descs(flushed - CH,
                                             (flushed - CH) & CH):
                            d.wait()
                    for d in flush_descs(flushed, flushed & CH):
                        d.start()

                flushed = jnp.where(do_flush, flushed + CH, flushed)
            return nc16, flushed

        nc16, flushed = lax.fori_loop(
            0, nch // 2, body,
            (jnp.zeros((L,), jnp.int32), jnp.int32(0)))
        nc = jnp.max(nc16)

        rem = nc - flushed
        flushed = pl.multiple_of(flushed, CH)
        off = pl.multiple_of(flushed & CH, CH)
        rem16 = jnp.full((L,), rem, jnp.int32)
        for g in range(CH // L):
            lane = jnp.arange(g * L, (g + 1) * L, dtype=jnp.int32)
            mf = lane >= rem16
            pos = jnp.full((L,), off, jnp.int32) + lane
            plsc.store_scatter(ccol, [pos], lane, mask=mf)
            plsc.store_scatter(cval, [pos], jnp.zeros((L,), jnp.float32),
                               mask=mf)
            plsc.store_scatter(csidx, [pos], jnp.full((L,), DUMP, jnp.int32),
                               mask=mf)

        @pl.when(flushed > 0)
        def _():
            for d in flush_descs(flushed - CH, (flushed - CH) & CH):
                d.wait()

        pltpu.sync_copy(ccol.at[pl.ds(off, CH)],
                        pcol_h.at[pl.ds(rbase + flushed, CH)])
        pltpu.sync_copy(cval.at[pl.ds(off, CH)],
                        pval_h.at[pl.ds(rbase + flushed, CH)])
        pltpu.sync_copy(csidx.at[pl.ds(off, CH)],
                        psidx_h.at[pl.ds(rbase + flushed, CH)])

        cnt16[...] = nc16
        pltpu.sync_copy(cnt16, counts_h.at[pl.ds(widx * L, L)])

    return part, reg


def _edge_scratch():
    return [
        pltpu.VMEM((CH,), jnp.int32),
        pltpu.VMEM((CH,), jnp.float32),
        pltpu.VMEM((CH,), jnp.int32),
        pltpu.VMEM((CH, D), jnp.float32),
        pltpu.SemaphoreType.DMA,
        pltpu.SemaphoreType.DMA,
        pltpu.SemaphoreType.DMA,
    ]


def _tile_chunk_count(counts_h, cnt16, widx):
    pltpu.sync_copy(counts_h.at[pl.ds(widx * L, L)], cnt16)
    cnt = jnp.max(cnt16[...])
    return jnp.maximum((cnt + CH - 1) >> 7, 1)


def _compact_edge_loop(rbase, nct, nch_max, pcol_h, pval_h, psidx_h,
                       x_h, acc, bufs):
    def issue_idx(i, b):
        off = rbase + i * CH
        pltpu.async_copy(pcol_h.at[pl.ds(off, CH)], b[0], b[4])
        pltpu.async_copy(pval_h.at[pl.ds(off, CH)], b[1], b[4])
        pltpu.async_copy(psidx_h.at[pl.ds(off, CH)], b[2], b[4])

    def wait_idx(i, b):
        off = rbase + i * CH
        pltpu.make_async_copy(pcol_h.at[pl.ds(off, CH)], b[0], b[4]).wait()
        pltpu.make_async_copy(pval_h.at[pl.ds(off, CH)], b[1], b[4]).wait()
        pltpu.make_async_copy(psidx_h.at[pl.ds(off, CH)], b[2], b[4]).wait()

    def issue_gather(b):
        pltpu.async_copy(x_h.at[b[0]], b[3], b[5])

    def wait_gather(b):
        pltpu.make_async_copy(x_h.at[b[0]], b[3], b[5]).wait()

    def issue_scatter(b):
        pltpu.async_copy(b[3], acc.at[b[2]], b[6], add=True)

    def wait_scatter(b):
        pltpu.make_async_copy(b[3], acc.at[b[2]], b[6]).wait()

    issue_idx(0, bufs[0])
    wait_idx(0, bufs[0])
    issue_gather(bufs[0])

    @pl.when(1 < nct)
    def _():
        issue_idx(1, bufs[1])

    def body(ii, _):
        for p in range(2):
            i = 2 * ii + p
            cur = bufs[p]
            nxt = bufs[1 - p]

            @pl.when(i < nct)
            def _():
                @pl.when(i + 1 < nct)
                def _():
                    wait_idx(i + 1, nxt)

                @pl.when(i >= 1)
                def _():
                    wait_scatter(nxt)

                @pl.when(i + 1 < nct)
                def _():
                    issue_gather(nxt)

                wait_gather(cur)
                _scale_rows(cur[3], cur[1])

                @pl.when(i + 2 < nct)
                def _():
                    issue_idx(i + 2, cur)

                issue_scatter(cur)
        return 0

    lax.fori_loop(0, nch_max // 2, body, 0)

    @pl.when((nct & 1) == 1)
    def _():
        wait_scatter(bufs[0])

    @pl.when((nct & 1) == 0)
    def _():
        wait_scatter(bufs[1])


def _make_spmm_compact(reg):
    nch_max = reg // CH

    @functools.partial(
        pl.kernel,
        out_type=jax.ShapeDtypeStruct((NU, D), jnp.float32),
        mesh=plsc.VectorSubcoreMesh(**_MESH),
        compiler_params=_SC_PARAMS,
        scratch_types=_edge_scratch() + _edge_scratch() + [
            pltpu.VMEM((L,), jnp.int32),
            pltpu.VMEM_SHARED((ACC_ROWS, D), jnp.float32),
        ],
    )
    def spmm(pcol_h, pval_h, psidx_h, counts_h, x_h, zeros_h, y_h, *scratch):
        bufs = (tuple(scratch[0:7]), tuple(scratch[7:14]))
        cnt16, acc = scratch[14], scratch[15]
        c = lax.axis_index("c")
        s = lax.axis_index("s")
        widx = c * NS + s

        _zero_acc_slice(zeros_h, acc, s)
        plsc.subcore_barrier()

        nct = _tile_chunk_count(counts_h, cnt16, widx)
        _compact_edge_loop(widx * reg, nct, nch_max,
                           pcol_h, pval_h, psidx_h, x_h, acc, bufs)
        plsc.subcore_barrier()

        start = s * 1568
        cbase = c * HALF

        @pl.when(s < 15)
        def _():
            pltpu.sync_copy(acc.at[pl.ds(start, 1568)],
                            y_h.at[pl.ds(cbase + start, 1568)])

        @pl.when(s == 15)
        def _():
            pltpu.sync_copy(acc.at[pl.ds(start, 1480)],
                            y_h.at[pl.ds(cbase + start, 1480)])

    return spmm


def _make_stage_b(reg):
    nch_max = reg // CH
    bpt = B // NS

    @functools.partial(
        pl.kernel,
        out_type=jax.ShapeDtypeStruct((NC * B,), jnp.float32),
        mesh=plsc.VectorSubcoreMesh(**_MESH),
        compiler_params=_SC_PARAMS,
        scratch_types=_edge_scratch() + _edge_scratch() + [
            pltpu.VMEM((L,), jnp.int32),
            pltpu.VMEM((bpt,), jnp.int32),
            pltpu.VMEM((bpt,), jnp.int32),
            pltpu.VMEM((CH,), jnp.int32),
            pltpu.VMEM((CH, D), jnp.float32),
            pltpu.VMEM((bpt,), jnp.float32),
            pltpu.VMEM_SHARED((ACC_ROWS, D), jnp.float32),
        ],
    )
    def stage_b(pcol_h, pval_h, psidx_h, counts_h, light_h, zeros_h,
                users_h, items_h, gpart_h, *scratch):
        bufs = (tuple(scratch[0:7]), tuple(scratch[7:14]))
        cnt16, users_v, items_v, iidx_v, vbuf, gout, acc = scratch[14:21]
        c = lax.axis_index("c")
        s = lax.axis_index("s")
        widx = c * NS + s
        ibase = c * HALF
        ubuf = bufs[0][3]
        sem_g = bufs[0][5]

        _zero_acc_slice(zeros_h, acc, s)
        plsc.subcore_barrier()

        nct = _tile_chunk_count(counts_h, cnt16, widx)
        _compact_edge_loop(widx * reg, nct, nch_max,
                           pcol_h, pval_h, psidx_h, light_h, acc, bufs)
        plsc.subcore_barrier()

        pltpu.sync_copy(users_h.at[pl.ds(s * bpt, bpt)], users_v)
        pltpu.sync_copy(items_h.at[pl.ds(s * bpt, bpt)], items_v)
        for half in range(bpt // CH):
            hsl = pl.ds(half * CH, CH)
            pltpu.async_copy(light_h.at[users_v.at[hsl]], ubuf, sem_g).wait()
            for g in range(CH // L):
                sl = pl.ds(half * CH + g * L, L)
                loc = items_v[sl] - ibase
                own = (loc >= 0) & (loc < HALF)
                fake = jnp.arange(g * L, (g + 1) * L, dtype=jnp.int32)
                iidx_v[pl.ds(g * L, L)] = jnp.where(own, loc, DUMP + fake)
            pltpu.sync_copy(acc.at[iidx_v], vbuf)
            for g in range(CH // L):
                rows16 = jnp.arange(g * L, (g + 1) * L, dtype=jnp.int32)

                def dot_body(d, a):
                    d16 = jnp.full((L,), d, jnp.int32)
                    u = plsc.load_gather(ubuf, [rows16, d16])
                    v = plsc.load_gather(vbuf, [rows16, d16])
                    return a + u * v

                acc16 = lax.fori_loop(0, D, dot_body,
                                      jnp.zeros((L,), jnp.float32))
                sl = pl.ds(half * CH + g * L, L)
                loc = items_v[sl] - ibase
                own = (loc >= 0) & (loc < HALF)
                gout[sl] = jnp.where(own, acc16, 0.0)
        pltpu.sync_copy(gout, gpart_h.at[pl.ds(c * B + s * bpt, bpt)])

    return stage_b


def _mean4(a, b, c, d):
    def body(a_r, b_r, c_r, d_r, o_r):
        o_r[...] = (a_r[...] + b_r[...] + c_r[...] + d_r[...]) * 0.25

    blk = 1000
    spec = pl.BlockSpec((blk, D), lambda i: (i, 0))
    return pl.pallas_call(
        body,
        out_shape=jax.ShapeDtypeStruct((NU, D), jnp.float32),
        grid=(NU // blk,),
        in_specs=[spec] * 4,
        out_specs=spec,
    )(a, b, c, d)


def _pad_edges(row, col, val, n_pad):
    n = row.shape[0]
    if n == n_pad:
        return row.astype(jnp.int32), col.astype(jnp.int32), val
    pz = n_pad - n
    row = jnp.concatenate([row.astype(jnp.int32), jnp.zeros((pz,), jnp.int32)])
    col = jnp.concatenate([col.astype(jnp.int32),
                           jnp.arange(pz, dtype=jnp.int32) % NU])
    val = jnp.concatenate([val, jnp.zeros((pz,), jnp.float32)])
    return row, col, val


@jax.jit
def kernel(users, items, uu_row, uu_col, uu_val, g_row, g_col, g_val, user_emb):
    n_uu = uu_row.shape[0]
    n_g = g_row.shape[0]
    unit = NS * CH * 2
    uu_pad = -(-n_uu // unit) * unit
    g_pad = -(-n_g // unit) * unit

    ur, uc, uv = _pad_edges(uu_row, uu_col, uu_val, uu_pad)
    gr, gc, gv = _pad_edges(g_row, g_col, g_val, g_pad)

    zeros_h = jnp.zeros((1576, D), jnp.float32)

    part_uu, reg_uu = _make_partition(uu_pad, bipartite=False)
    pcol_u, pval_u, psidx_u, cnt_u = part_uu(ur, uc, uv)

    spmm = _make_spmm_compact(reg_uu)
    x0 = user_emb
    x1 = spmm(pcol_u, pval_u, psidx_u, cnt_u, x0, zeros_h)
    x2 = spmm(pcol_u, pval_u, psidx_u, cnt_u, x1, zeros_h)
    x3 = spmm(pcol_u, pval_u, psidx_u, cnt_u, x2, zeros_h)
    light = _mean4(x0, x1, x2, x3)

    part_g, reg_g = _make_partition(g_pad, bipartite=True)
    pcol_g, pval_g, psidx_g, cnt_g = part_g(gr, gc, gv)

    stage_b = _make_stage_b(reg_g)
    gpart = stage_b(pcol_g, pval_g, psidx_g, cnt_g, light, zeros_h,
                    users.astype(jnp.int32), items.astype(jnp.int32))
    return gpart[:B] + gpart[B:]
